# baseline (speedup 1.0000x reference)
import functools

import jax
import jax.numpy as jnp
from jax import lax
from jax.experimental import pallas as pl
from jax.experimental.pallas import tpu as pltpu
from jax.experimental.pallas import tpu_sc as plsc

N_NODES = 10000
D = 128
N_EDGES = 320000

NC = 2
NS = 16
NW = NC * NS

K = 128
CHUNKS_PER_W = 80
EDGES_PER_W = CHUNKS_PER_W * K
E_PAD = EDGES_PER_W * NW
N_CHUNK_ROWS = E_PAD // K

ROWS_PER_TILE = 640
N_PAD = ROWS_PER_TILE * NS

_mesh = plsc.VectorSubcoreMesh(
    core_axis_name="c", subcore_axis_name="s", num_cores=NC, num_subcores=NS)


@functools.partial(
    pl.kernel,
    out_type=jax.ShapeDtypeStruct((NC, N_PAD, D), jnp.float32),
    mesh=_mesh,
    scratch_types=[
        pltpu.VMEM((K,), jnp.int32),
        pltpu.VMEM((K,), jnp.int32),
        pltpu.VMEM((K,), jnp.int32),
        pltpu.VMEM((K,), jnp.int32),
        pltpu.VMEM((K, D), jnp.float32),
        pltpu.VMEM((K, D), jnp.float32),
        pltpu.VMEM_SHARED((N_PAD, D), jnp.float32),
        pltpu.SemaphoreType.DMA,
        pltpu.SemaphoreType.DMA,
        pltpu.SemaphoreType.DMA,
        pltpu.SemaphoreType.DMA,
        pltpu.SemaphoreType.DMA,
        pltpu.SemaphoreType.DMA,
        pltpu.SemaphoreType.DMA,
        pltpu.SemaphoreType.DMA,
    ],
)
def _sc_scatter(x_hbm, src_hbm, dst_hbm, out_hbm,
                sidx0, sidx1, didx0, didx1, rows0, rows1, acc,
                iss0, iss1, ids0, ids1, gs0, gs1, ss0, ss1):
    c = lax.axis_index("c")
    s = lax.axis_index("s")

    zv = jnp.zeros((16,), jnp.float32)

    def zbody(i, carry):
        rows0[i // 8, pl.ds((i % 8) * 16, 16)] = zv
        return carry

    lax.fori_loop(0, K * D // 16, zbody, 0)
    row0 = s * ROWS_PER_TILE
    for j in range(ROWS_PER_TILE // K):
        pltpu.sync_copy(rows0, acc.at[pl.ds(row0 + j * K, K)])
    plsc.subcore_barrier()

    wid = s * NC + c
    crow = wid * CHUNKS_PER_W

    def gather_c(i, sbuf, rbuf, sem):
        pltpu.async_copy(x_hbm.at[sbuf], rbuf, sem)

    def wait_gather(sbuf, rbuf, sem):
        pltpu.make_async_copy(x_hbm.at[sbuf], rbuf, sem).wait()

    def scatter_c(dbuf, rbuf, sem):
        pltpu.async_copy(rbuf, acc.at[dbuf], sem, add=True)

    def wait_scatter(dbuf, rbuf, sem):
        pltpu.make_async_copy(rbuf, acc.at[dbuf], sem).wait()

    def idx_copy(hbm, i, buf, sem):
        pltpu.async_copy(hbm.at[crow + i], buf, sem)

    def wait_idx(hbm, i, buf, sem):
        pltpu.make_async_copy(hbm.at[crow + i], buf, sem).wait()

    pltpu.sync_copy(src_hbm.at[crow + 0], sidx0)
    pltpu.sync_copy(src_hbm.at[crow + 1], sidx1)
    idx_copy(dst_hbm, 0, didx0, ids0)
    idx_copy(dst_hbm, 1, didx1, ids1)
    gather_c(0, sidx0, rows0, gs0)
    gather_c(1, sidx1, rows1, gs1)

    def body(p, carry):
        i0 = 2 * p
        i1 = i0 + 1
        wait_gather(sidx0, rows0, gs0)
        wait_idx(dst_hbm, i0, didx0, ids0)
        scatter_c(didx0, rows0, ss0)
        @pl.when(i0 + 2 < CHUNKS_PER_W)
        def _():
            idx_copy(src_hbm, i0 + 2, sidx0, iss0)
        wait_scatter(didx0, rows0, ss0)
        @pl.when(i0 + 2 < CHUNKS_PER_W)
        def _():
            idx_copy(dst_hbm, i0 + 2, didx0, ids0)
            wait_idx(src_hbm, i0 + 2, sidx0, iss0)
            gather_c(i0 + 2, sidx0, rows0, gs0)
        wait_gather(sidx1, rows1, gs1)
        wait_idx(dst_hbm, i1, didx1, ids1)
        scatter_c(didx1, rows1, ss1)
        @pl.when(i1 + 2 < CHUNKS_PER_W)
        def _():
            idx_copy(src_hbm, i1 + 2, sidx1, iss1)
        wait_scatter(didx1, rows1, ss1)
        @pl.when(i1 + 2 < CHUNKS_PER_W)
        def _():
            idx_copy(dst_hbm, i1 + 2, didx1, ids1)
            wait_idx(src_hbm, i1 + 2, sidx1, iss1)
            gather_c(i1 + 2, sidx1, rows1, gs1)
        return carry

    lax.fori_loop(0, CHUNKS_PER_W // 2, body, 0)
    plsc.subcore_barrier()

    pltpu.sync_copy(acc.at[pl.ds(row0, ROWS_PER_TILE)],
                    out_hbm.at[c, pl.ds(row0, ROWS_PER_TILE)])


def _combine_body(a0_ref, a1_ref, x_ref, w_ref, o_ref):
    t = a0_ref[0] + a1_ref[0] + x_ref[...]
    o_ref[...] = jnp.dot(t, w_ref[...], preferred_element_type=jnp.float32)


_R_BLK = 400


def _combine(agg, x, W):
    return pl.pallas_call(
        _combine_body,
        grid=(N_NODES // _R_BLK,),
        in_specs=[
            pl.BlockSpec((1, _R_BLK, D), lambda i: (0, i, 0)),
            pl.BlockSpec((1, _R_BLK, D), lambda i: (1, i, 0)),
            pl.BlockSpec((_R_BLK, D), lambda i: (i, 0)),
            pl.BlockSpec((D, D), lambda i: (0, 0)),
        ],
        out_specs=pl.BlockSpec((_R_BLK, D), lambda i: (i, 0)),
        out_shape=jax.ShapeDtypeStruct((N_NODES, D), jnp.float32),
    )(agg, agg, x, W)


def kernel(x, edge_index, W):
    src = edge_index[0].astype(jnp.int32)
    dst = edge_index[1].astype(jnp.int32)
    pad = E_PAD - N_EDGES
    src_p = jnp.concatenate([src, jnp.zeros((pad,), jnp.int32)])
    dst_p = jnp.concatenate([dst, jnp.full((pad,), N_NODES, jnp.int32)])
    src_p = src_p.reshape(N_CHUNK_ROWS, K)
    dst_p = dst_p.reshape(N_CHUNK_ROWS, K)
    agg = _sc_scatter(x, src_p, dst_p)
    return _combine(agg, x, W)
